# baseline (speedup 1.0000x reference)
import functools

import jax
import jax.numpy as jnp
from jax import lax
from jax.experimental import pallas as pl
from jax.experimental.pallas import tpu as pltpu
from jax.experimental.pallas import tpu_sc as plsc

_D = 64
_NEG = 5
_NC = 2
_NS = 16
_NW = _NC * _NS
_CHUNK = 64
_SUB = 64
_L = 16


def _sc_scores(pos_u, pos_v, neg_flat, u_emb, v_emb):
  B = pos_u.shape[0]
  per_w = B // _NW
  n_chunks = per_w // _CHUNK
  mesh = plsc.VectorSubcoreMesh(core_axis_name="c", subcore_axis_name="s")

  @functools.partial(
      pl.kernel,
      out_type=(
          jax.ShapeDtypeStruct((B,), jnp.float32),
          jax.ShapeDtypeStruct((B * _NEG,), jnp.float32),
      ),
      mesh=mesh,
      compiler_params=pltpu.CompilerParams(
          needs_layout_passes=False, use_tc_tiling_on_sc=True),
      scratch_types=[
          pltpu.VMEM((per_w,), jnp.int32),
          pltpu.VMEM((per_w,), jnp.int32),
          pltpu.VMEM((per_w * _NEG,), jnp.int32),
          pltpu.VMEM((_CHUNK, _D), jnp.float32),
          pltpu.VMEM((_CHUNK, _D), jnp.float32),
          pltpu.VMEM((_CHUNK, _D), jnp.float32),
          pltpu.VMEM((_CHUNK, _D), jnp.float32),
          pltpu.VMEM((_CHUNK * _NEG, _D), jnp.float32),
          pltpu.VMEM((_CHUNK * _NEG, _D), jnp.float32),
          pltpu.VMEM(((1 + _NEG) * _L * _SUB,), jnp.float32),
          pltpu.VMEM((_CHUNK,), jnp.float32),
          pltpu.VMEM((_CHUNK * _NEG,), jnp.float32),
          pltpu.SemaphoreType.DMA,
          pltpu.SemaphoreType.DMA,
      ],
  )
  def k(pos_u_h, pos_v_h, neg_h, u_h, v_h, out_pos, out_neg,
        idx_u, idx_v, idx_n,
        u0, u1, v0, v1, n0, n1, partials, ps, ns, sem0, sem1):
    wid = lax.axis_index("s") * _NC + lax.axis_index("c")
    base = wid * per_w
    pltpu.sync_copy(pos_u_h.at[pl.ds(base, per_w)], idx_u)
    pltpu.sync_copy(pos_v_h.at[pl.ds(base, per_w)], idx_v)
    pltpu.sync_copy(neg_h.at[pl.ds(base * _NEG, per_w * _NEG)], idx_n)

    ubufs, vbufs, nbufs, sems = (u0, u1), (v0, v1), (n0, n1), (sem0, sem1)

    def start(c, b):
      @plsc.parallel_loop(0, _CHUNK // _L)
      def _uv(g, b=b, c=c):
        ru = idx_u[pl.ds(c * _CHUNK + g * _L, _L)]
        rv = idx_v[pl.ds(c * _CHUNK + g * _L, _L)]
        for j in range(_L):
          pltpu.async_copy(
              u_h.at[pl.ds(ru[j], 1)],
              ubufs[b].at[pl.ds(g * _L + j, 1)], sems[b])
          pltpu.async_copy(
              v_h.at[pl.ds(rv[j], 1)],
              vbufs[b].at[pl.ds(g * _L + j, 1)], sems[b])

      @plsc.parallel_loop(0, _CHUNK * _NEG // _L)
      def _nn(g, b=b, c=c):
        rn = idx_n[pl.ds(c * _CHUNK * _NEG + g * _L, _L)]
        for j in range(_L):
          pltpu.async_copy(
              v_h.at[pl.ds(rn[j], 1)],
              nbufs[b].at[pl.ds(g * _L + j, 1)], sems[b])

    def wait(b):
      pltpu.make_async_copy(
          u_h.at[pl.ds(0, _CHUNK)], ubufs[b], sems[b]).wait()
      pltpu.make_async_copy(
          u_h.at[pl.ds(0, _CHUNK)], vbufs[b], sems[b]).wait()
      pltpu.make_async_copy(
          u_h.at[pl.ds(0, _CHUNK * _NEG)], nbufs[b], sems[b]).wait()

    iota = lax.iota(jnp.int32, _L)
    bases = [s * (_L * _SUB) + iota * _SUB for s in range(1 + _NEG)]
    nq = _D // _L

    def compute(c, b):
      ub, vb, nb = ubufs[b], vbufs[b], nbufs[b]

      for half in range(_CHUNK // _SUB):
        h0 = half * _SUB

        @plsc.parallel_loop(0, _SUB, unroll=4)
        def p_body(p, h0=h0):
          pa = h0 + p
          u4 = [ub[pa, pl.ds(q * _L, _L)] for q in range(nq)]
          v4 = [vb[pa, pl.ds(q * _L, _L)] for q in range(nq)]
          s = (u4[0] * v4[0] + u4[1] * v4[1]) + (u4[2] * v4[2] + u4[3] * v4[3])
          plsc.store_scatter(partials, [bases[0] + p], s)
          p5 = pa * _NEG
          for n in range(_NEG):
            n4 = [nb[p5 + n, pl.ds(q * _L, _L)] for q in range(nq)]
            sn = (n4[0] * u4[0] + n4[1] * u4[1]) + (n4[2] * u4[2] + n4[3] * u4[3])
            plsc.store_scatter(partials, [bases[1 + n] + p], sn)

        @plsc.parallel_loop(0, _SUB // _L, unroll=2)
        def g_body(g, h0=h0):
          for s in range(1 + _NEG):
            off = s * (_L * _SUB) + g * _L
            vals = [partials[pl.ds(off + l * _SUB, _L)] for l in range(_L)]
            while len(vals) > 1:
              vals = [vals[i] + vals[i + 1] for i in range(0, len(vals), 2)]
            if s == 0:
              ps[pl.ds(h0 + g * _L, _L)] = vals[0]
            else:
              ns[pl.ds(s * _CHUNK - _CHUNK + h0 + g * _L, _L)] = vals[0]

      pltpu.sync_copy(ps, out_pos.at[pl.ds(base + c * _CHUNK, _CHUNK)])
      pltpu.sync_copy(
          ns, out_neg.at[pl.ds((base + c * _CHUNK) * _NEG, _CHUNK * _NEG)])

    start(0, 0)

    def ring(g, carry):
      c0 = 2 * g
      start(c0 + 1, 1)
      wait(0)
      compute(c0, 0)

      @pl.when(g < n_chunks // 2 - 1)
      def _():
        start(c0 + 2, 0)

      wait(1)
      compute(c0 + 1, 1)
      return carry

    lax.fori_loop(0, n_chunks // 2, ring, 0)

  return k(pos_u, pos_v, neg_flat, u_emb, v_emb)


def _tc_loss(pos_s, neg_s, batch):
  def body(ps_ref, ns_ref, out_ref):
    s = jnp.clip(ps_ref[...], -10.0, 10.0)
    t = jnp.clip(ns_ref[...], -10.0, 10.0)
    tot = jnp.sum(jnp.log1p(jnp.exp(-s))) + jnp.sum(jnp.log1p(jnp.exp(t)))
    out_ref[...] = jnp.reshape(tot * (1.0 / batch), (1, 1))

  out = pl.pallas_call(
      body,
      out_shape=jax.ShapeDtypeStruct((1, 1), jnp.float32),
  )(pos_s.reshape(batch // 128, 128), neg_s.reshape(batch * _NEG // 128, 128))
  return out[0, 0]


def kernel(pos_u, pos_v, neg_v, u_emb, v_emb):
  batch = pos_u.shape[0]
  pos_s, neg_s = _sc_scores(pos_u, pos_v, neg_v.reshape(-1), u_emb, v_emb)
  return _tc_loss(pos_s, neg_s, batch)


# speedup vs baseline: 1.0005x; 1.0005x over previous
import functools

import jax
import jax.numpy as jnp
from jax import lax
from jax.experimental import pallas as pl
from jax.experimental.pallas import tpu as pltpu
from jax.experimental.pallas import tpu_sc as plsc

_D = 64
_NEG = 5
_NC = 2
_NS = 16
_NW = _NC * _NS
_CHUNK = 64
_SUB = 64
_L = 16


def _sc_scores(pos_u, pos_v, neg_flat, u_emb, v_emb):
  B = pos_u.shape[0]
  per_w = B // _NW
  n_chunks = per_w // _CHUNK
  mesh = plsc.VectorSubcoreMesh(core_axis_name="c", subcore_axis_name="s")

  @functools.partial(
      pl.kernel,
      out_type=(
          jax.ShapeDtypeStruct((B,), jnp.float32),
          jax.ShapeDtypeStruct((B * _NEG,), jnp.float32),
      ),
      mesh=mesh,
      compiler_params=pltpu.CompilerParams(
          needs_layout_passes=False, use_tc_tiling_on_sc=True),
      scratch_types=[
          pltpu.VMEM((per_w,), jnp.int32),
          pltpu.VMEM((per_w,), jnp.int32),
          pltpu.VMEM((per_w * _NEG,), jnp.int32),
          pltpu.VMEM((_CHUNK, _D), jnp.float32),
          pltpu.VMEM((_CHUNK, _D), jnp.float32),
          pltpu.VMEM((_CHUNK, _D), jnp.float32),
          pltpu.VMEM((_CHUNK, _D), jnp.float32),
          pltpu.VMEM((_CHUNK * _NEG, _D), jnp.float32),
          pltpu.VMEM((_CHUNK * _NEG, _D), jnp.float32),
          pltpu.VMEM(((1 + _NEG) * _L * _SUB,), jnp.float32),
          pltpu.VMEM((_CHUNK,), jnp.float32),
          pltpu.VMEM((_CHUNK * _NEG,), jnp.float32),
          pltpu.SemaphoreType.DMA,
          pltpu.SemaphoreType.DMA,
          pltpu.SemaphoreType.DMA,
          pltpu.SemaphoreType.DMA,
      ],
  )
  def k(pos_u_h, pos_v_h, neg_h, u_h, v_h, out_pos, out_neg,
        idx_u, idx_v, idx_n,
        u0, u1, v0, v1, n0, n1, partials, ps, ns,
        sem0, sem1, nsem0, nsem1):
    wid = lax.axis_index("s") * _NC + lax.axis_index("c")
    base = wid * per_w
    pltpu.sync_copy(pos_u_h.at[pl.ds(base, per_w)], idx_u)
    pltpu.sync_copy(pos_v_h.at[pl.ds(base, per_w)], idx_v)
    pltpu.sync_copy(neg_h.at[pl.ds(base * _NEG, per_w * _NEG)], idx_n)

    ubufs, vbufs, nbufs, sems = (u0, u1), (v0, v1), (n0, n1), (sem0, sem1)
    nsems = (nsem0, nsem1)

    def start(c, b):
      @plsc.parallel_loop(0, _CHUNK // _L)
      def _uv(g, b=b, c=c):
        ru = idx_u[pl.ds(c * _CHUNK + g * _L, _L)]
        rv = idx_v[pl.ds(c * _CHUNK + g * _L, _L)]
        for j in range(_L):
          pltpu.async_copy(
              u_h.at[pl.ds(ru[j], 1)],
              ubufs[b].at[pl.ds(g * _L + j, 1)], sems[b])
          pltpu.async_copy(
              v_h.at[pl.ds(rv[j], 1)],
              vbufs[b].at[pl.ds(g * _L + j, 1)], sems[b])

      @plsc.parallel_loop(0, _CHUNK * _NEG // _L)
      def _nn(g, b=b, c=c):
        rn = idx_n[pl.ds(c * _CHUNK * _NEG + g * _L, _L)]
        for j in range(_L):
          pltpu.async_copy(
              v_h.at[pl.ds(rn[j], 1)],
              nbufs[b].at[pl.ds(g * _L + j, 1)], nsems[b])

    def wait(b):
      pltpu.make_async_copy(
          u_h.at[pl.ds(0, _CHUNK)], ubufs[b], sems[b]).wait()
      pltpu.make_async_copy(
          u_h.at[pl.ds(0, _CHUNK)], vbufs[b], sems[b]).wait()
      pltpu.make_async_copy(
          u_h.at[pl.ds(0, _CHUNK * _NEG)], nbufs[b], nsems[b]).wait()

    iota = lax.iota(jnp.int32, _L)
    bases = [s * (_L * _SUB) + iota * _SUB for s in range(1 + _NEG)]
    nq = _D // _L

    def compute(c, b):
      ub, vb, nb = ubufs[b], vbufs[b], nbufs[b]

      for half in range(_CHUNK // _SUB):
        h0 = half * _SUB

        @plsc.parallel_loop(0, _SUB, unroll=4)
        def p_body(p, h0=h0):
          pa = h0 + p
          u4 = [ub[pa, pl.ds(q * _L, _L)] for q in range(nq)]
          v4 = [vb[pa, pl.ds(q * _L, _L)] for q in range(nq)]
          s = (u4[0] * v4[0] + u4[1] * v4[1]) + (u4[2] * v4[2] + u4[3] * v4[3])
          plsc.store_scatter(partials, [bases[0] + p], s)
          p5 = pa * _NEG
          for n in range(_NEG):
            n4 = [nb[p5 + n, pl.ds(q * _L, _L)] for q in range(nq)]
            sn = (n4[0] * u4[0] + n4[1] * u4[1]) + (n4[2] * u4[2] + n4[3] * u4[3])
            plsc.store_scatter(partials, [bases[1 + n] + p], sn)

        @plsc.parallel_loop(0, _SUB // _L, unroll=2)
        def g_body(g, h0=h0):
          for s in range(1 + _NEG):
            off = s * (_L * _SUB) + g * _L
            vals = [partials[pl.ds(off + l * _SUB, _L)] for l in range(_L)]
            while len(vals) > 1:
              vals = [vals[i] + vals[i + 1] for i in range(0, len(vals), 2)]
            if s == 0:
              ps[pl.ds(h0 + g * _L, _L)] = vals[0]
            else:
              ns[pl.ds(s * _CHUNK - _CHUNK + h0 + g * _L, _L)] = vals[0]

      pltpu.sync_copy(ps, out_pos.at[pl.ds(base + c * _CHUNK, _CHUNK)])
      pltpu.sync_copy(
          ns, out_neg.at[pl.ds((base + c * _CHUNK) * _NEG, _CHUNK * _NEG)])

    start(0, 0)

    def ring(g, carry):
      c0 = 2 * g
      start(c0 + 1, 1)
      wait(0)
      compute(c0, 0)

      @pl.when(g < n_chunks // 2 - 1)
      def _():
        start(c0 + 2, 0)

      wait(1)
      compute(c0 + 1, 1)
      return carry

    lax.fori_loop(0, n_chunks // 2, ring, 0)

  return k(pos_u, pos_v, neg_flat, u_emb, v_emb)


def _tc_loss(pos_s, neg_s, batch):
  def body(ps_ref, ns_ref, out_ref):
    s = jnp.clip(ps_ref[...], -10.0, 10.0)
    t = jnp.clip(ns_ref[...], -10.0, 10.0)
    tot = jnp.sum(jnp.log1p(jnp.exp(-s))) + jnp.sum(jnp.log1p(jnp.exp(t)))
    out_ref[...] = jnp.reshape(tot * (1.0 / batch), (1, 1))

  out = pl.pallas_call(
      body,
      out_shape=jax.ShapeDtypeStruct((1, 1), jnp.float32),
  )(pos_s.reshape(batch // 128, 128), neg_s.reshape(batch * _NEG // 128, 128))
  return out[0, 0]


def kernel(pos_u, pos_v, neg_v, u_emb, v_emb):
  batch = pos_u.shape[0]
  pos_s, neg_s = _sc_scores(pos_u, pos_v, neg_v.reshape(-1), u_emb, v_emb)
  return _tc_loss(pos_s, neg_s, batch)
